# baseline (speedup 1.0000x reference)
import functools

import jax
import jax.numpy as jnp
from jax import lax
from jax.experimental import pallas as pl
from jax.experimental.pallas import tpu as pltpu
from jax.experimental.pallas import tpu_sc as plsc

F32 = jnp.float32
LAT = 128
N_NODES = 10000
NCORE = 2
NSUB = 16
NW = NCORE * NSUB
CH = 128
BR = 2000

_EKEYS = ('coarse', 'mesh', 'world')


def _ln_apply(y, g, b):
    m = jnp.mean(y, axis=-1, keepdims=True)
    v = jnp.mean((y - m) ** 2, axis=-1, keepdims=True)
    return (y - m) * lax.rsqrt(v + 1e-5) * g + b


def _unpack_mlp(p):
    ls = p['layers']
    out = []
    for l in ls:
        out.append(l['W'])
        out.append(l['b'].reshape(1, -1))
    if 'ln' in p:
        out.append(p['ln']['g'].reshape(1, -1))
        out.append(p['ln']['b'].reshape(1, -1))
    return out


def _wspec(shape):
    return pl.BlockSpec(shape, lambda i: (0, 0))


def _mlp_ln(x, w0, b0, w1, b1, w2, b2, g, bb):
    n, din = x.shape

    def body(x_r, w0_r, b0_r, w1_r, b1_r, w2_r, b2_r, g_r, bb_r, o_r):
        h = jnp.maximum(jnp.dot(x_r[...], w0_r[...], preferred_element_type=F32) + b0_r[...], 0.0)
        h = jnp.maximum(jnp.dot(h, w1_r[...], preferred_element_type=F32) + b1_r[...], 0.0)
        y = jnp.dot(h, w2_r[...], preferred_element_type=F32) + b2_r[...]
        o_r[...] = _ln_apply(y, g_r[...], bb_r[...])

    return pl.pallas_call(
        body,
        grid=(n // BR,),
        in_specs=[
            pl.BlockSpec((BR, din), lambda i: (i, 0)),
            _wspec((din, LAT)), _wspec((1, LAT)),
            _wspec((LAT, LAT)), _wspec((1, LAT)),
            _wspec((LAT, LAT)), _wspec((1, LAT)),
            _wspec((1, LAT)), _wspec((1, LAT)),
        ],
        out_specs=pl.BlockSpec((BR, LAT), lambda i: (i, 0)),
        out_shape=jax.ShapeDtypeStruct((n, LAT), F32),
    )(x, w0, b0, w1, b1, w2, b2, g, bb)


def _proj6(x, ws):
    n = x.shape[0]

    def body(x_r, *refs):
        xv = x_r[...]
        for k in range(6):
            refs[6 + k][...] = jnp.dot(xv, refs[k][...], preferred_element_type=F32)

    return pl.pallas_call(
        body,
        grid=(n // BR,),
        in_specs=[pl.BlockSpec((BR, LAT), lambda i: (i, 0))] + [_wspec((LAT, LAT))] * 6,
        out_specs=[pl.BlockSpec((BR, LAT), lambda i: (i, 0))] * 6,
        out_shape=[jax.ShapeDtypeStruct((n, LAT), F32)] * 6,
    )(x, *ws)


def _edge_update_enc(gsum, raw_t, encp, w0f, b0, w1, b1, w2, b2, g, bb):
    din, e = raw_t.shape
    ew0, eb0, ew1, eb1, ew2, eb2, eg, ebb = encp

    br = 640

    def body(gs_r, raw_r, ew0_r, eb0_r, ew1_r, eb1_r, ew2_r, eb2_r, eg_r, ebb_r,
             w0_r, b0_r, w1_r, b1_r, w2_r, b2_r, g_r, bb_r, nf_o, fn_o):
        h = lax.dot_general(raw_r[...], ew0_r[...], (((0,), (0,)), ((), ())),
                            preferred_element_type=F32)
        h = jnp.maximum(h + eb0_r[...], 0.0)
        h = jnp.maximum(jnp.dot(h, ew1_r[...], preferred_element_type=F32) + eb1_r[...], 0.0)
        y = jnp.dot(h, ew2_r[...], preferred_element_type=F32) + eb2_r[...]
        fv = _ln_apply(y, eg_r[...], ebb_r[...])
        h = gs_r[...] + jnp.dot(fv, w0_r[...], preferred_element_type=F32) + b0_r[...]
        h = jnp.maximum(h, 0.0)
        h = jnp.maximum(jnp.dot(h, w1_r[...], preferred_element_type=F32) + b1_r[...], 0.0)
        y = jnp.dot(h, w2_r[...], preferred_element_type=F32) + b2_r[...]
        nf = _ln_apply(y, g_r[...], bb_r[...])
        nf_o[...] = nf
        fn_o[...] = fv + nf

    return pl.pallas_call(
        body,
        grid=(e // br,),
        in_specs=[pl.BlockSpec((br, LAT), lambda i: (i, 0)),
                  pl.BlockSpec((din, br), lambda i: (0, i)),
                  _wspec((din, LAT)), _wspec((1, LAT)),
                  _wspec((LAT, LAT)), _wspec((1, LAT)),
                  _wspec((LAT, LAT)), _wspec((1, LAT)),
                  _wspec((1, LAT)), _wspec((1, LAT)),
                  _wspec((LAT, LAT)), _wspec((1, LAT)),
                  _wspec((LAT, LAT)), _wspec((1, LAT)),
                  _wspec((LAT, LAT)), _wspec((1, LAT)),
                  _wspec((1, LAT)), _wspec((1, LAT))],
        out_specs=[pl.BlockSpec((br, LAT), lambda i: (i, 0))] * 2,
        out_shape=[jax.ShapeDtypeStruct((e, LAT), F32)] * 2,
    )(gsum, raw_t, *encp, w0f, b0, w1, b1, w2, b2, g, bb)


def _edge_update(gsum, feats, w0f, b0, w1, b1, w2, b2, g, bb, want_fnew):
    e = feats.shape[0]

    def body(gs_r, f_r, w0_r, b0_r, w1_r, b1_r, w2_r, b2_r, g_r, bb_r, *outs):
        fv = f_r[...]
        h = gs_r[...] + jnp.dot(fv, w0_r[...], preferred_element_type=F32) + b0_r[...]
        h = jnp.maximum(h, 0.0)
        h = jnp.maximum(jnp.dot(h, w1_r[...], preferred_element_type=F32) + b1_r[...], 0.0)
        y = jnp.dot(h, w2_r[...], preferred_element_type=F32) + b2_r[...]
        nf = _ln_apply(y, g_r[...], bb_r[...])
        outs[0][...] = nf
        if want_fnew:
            outs[1][...] = fv + nf

    n_out = 2 if want_fnew else 1
    return pl.pallas_call(
        body,
        grid=(e // BR,),
        in_specs=[pl.BlockSpec((BR, LAT), lambda i: (i, 0))] * 2 + [
            _wspec((LAT, LAT)), _wspec((1, LAT)),
            _wspec((LAT, LAT)), _wspec((1, LAT)),
            _wspec((LAT, LAT)), _wspec((1, LAT)),
            _wspec((1, LAT)), _wspec((1, LAT)),
        ],
        out_specs=[pl.BlockSpec((BR, LAT), lambda i: (i, 0))] * n_out,
        out_shape=[jax.ShapeDtypeStruct((e, LAT), F32)] * n_out,
    )(gsum, feats, w0f, b0, w1, b1, w2, b2, g, bb)


def _node_update(x, aggs, wx, wc, wm, ww, b0, w1, b1, w2, b2, g, bb):
    n = x.shape[0]
    nb = N_NODES // BR

    def body(x_r, c0_r, c1_r, m0_r, m1_r, v0_r, v1_r,
             wx_r, wc_r, wm_r, ww_r, b0_r, w1_r, b1_r, w2_r, b2_r, g_r, bb_r, o_r):
        xv = x_r[...]
        h = jnp.dot(xv, wx_r[...], preferred_element_type=F32)
        h += jnp.dot(c0_r[...] + c1_r[...], wc_r[...], preferred_element_type=F32)
        h += jnp.dot(m0_r[...] + m1_r[...], wm_r[...], preferred_element_type=F32)
        h += jnp.dot(v0_r[...] + v1_r[...], ww_r[...], preferred_element_type=F32)
        h = jnp.maximum(h + b0_r[...], 0.0)
        h = jnp.maximum(jnp.dot(h, w1_r[...], preferred_element_type=F32) + b1_r[...], 0.0)
        y = jnp.dot(h, w2_r[...], preferred_element_type=F32) + b2_r[...]
        o_r[...] = xv + _ln_apply(y, g_r[...], bb_r[...])

    part_specs = []
    for _ in range(3):
        part_specs.append(pl.BlockSpec((BR, LAT), lambda i: (i, 0)))
        part_specs.append(pl.BlockSpec((BR, LAT), lambda i, _nb=nb: (i + _nb, 0)))
    return pl.pallas_call(
        body,
        grid=(n // BR,),
        in_specs=[pl.BlockSpec((BR, LAT), lambda i: (i, 0))] + part_specs + [
            _wspec((LAT, LAT)), _wspec((LAT, LAT)), _wspec((LAT, LAT)), _wspec((LAT, LAT)),
            _wspec((1, LAT)),
            _wspec((LAT, LAT)), _wspec((1, LAT)),
            _wspec((LAT, LAT)), _wspec((1, LAT)),
            _wspec((1, LAT)), _wspec((1, LAT)),
        ],
        out_specs=pl.BlockSpec((BR, LAT), lambda i: (i, 0)),
        out_shape=jax.ShapeDtypeStruct((n, LAT), F32),
    )(x, aggs[0], aggs[0], aggs[1], aggs[1], aggs[2], aggs[2],
      wx, wc, wm, ww, b0, w1, b1, w2, b2, g, bb)


def _decode(x, w0, b0, w1, b1, w2, b2):
    n = x.shape[0]
    dout = w2.shape[1]

    def body(x_r, w0_r, b0_r, w1_r, b1_r, w2_r, b2_r, o_r):
        h = jnp.maximum(jnp.dot(x_r[...], w0_r[...], preferred_element_type=F32) + b0_r[...], 0.0)
        h = jnp.maximum(jnp.dot(h, w1_r[...], preferred_element_type=F32) + b1_r[...], 0.0)
        o_r[...] = jnp.dot(h, w2_r[...], preferred_element_type=F32) + b2_r[...]

    return pl.pallas_call(
        body,
        grid=(n // BR,),
        in_specs=[
            pl.BlockSpec((BR, LAT), lambda i: (i, 0)),
            _wspec((LAT, LAT)), _wspec((1, LAT)),
            _wspec((LAT, LAT)), _wspec((1, LAT)),
            _wspec((LAT, dout)), _wspec((1, dout)),
        ],
        out_specs=pl.BlockSpec((BR, dout), lambda i: (i, 0)),
        out_shape=jax.ShapeDtypeStruct((n, dout), F32),
    )(x, w0, b0, w1, b1, w2, b2)


@functools.lru_cache(maxsize=None)
def _sc_gather_fn(e, tok_shape):
    nchunk = e // CH
    mesh = plsc.VectorSubcoreMesh(core_axis_name="c", subcore_axis_name="s")

    def body(tok_h, xs_h, xd_h, src_h, dst_h, gs_h,
             ix0, ix1, id0, id1, a0, a1, b0, b1,
             isem0, isem1, gsem0, gsem1, ssem0, ssem1):
        del tok_h
        cc = lax.axis_index("c")
        ss = lax.axis_index("s")
        w = ss * NCORE + cc
        nt = (nchunk - w + NW - 1) // NW

        ixs = (ix0, ix1)
        ids = (id0, id1)
        avs = (a0, a1)
        bvs = (b0, b1)
        isems = (isem0, isem1)
        gsems = (gsem0, gsem1)
        ssems = (ssem0, ssem1)

        def idx_start(t, s):
            base = (w + t * NW) * CH
            pltpu.async_copy(src_h.at[pl.ds(base, CH)], ixs[s], isems[s])
            pltpu.async_copy(dst_h.at[pl.ds(base, CH)], ids[s], isems[s])

        def idx_wait(s):
            pltpu.make_async_copy(src_h.at[pl.ds(0, CH)], ixs[s], isems[s]).wait()
            pltpu.make_async_copy(dst_h.at[pl.ds(0, CH)], ids[s], isems[s]).wait()

        def gat_start(s):
            pltpu.async_copy(xs_h.at[ixs[s]], avs[s], gsems[s])
            pltpu.async_copy(xd_h.at[ids[s]], bvs[s], gsems[s])

        def gat_wait(s):
            pltpu.make_async_copy(xs_h.at[ixs[s]], avs[s], gsems[s]).wait()
            pltpu.make_async_copy(xd_h.at[ids[s]], bvs[s], gsems[s]).wait()

        def st_start(t, s):
            base = (w + t * NW) * CH
            pltpu.async_copy(avs[s], gs_h.at[pl.ds(base, CH)], ssems[s])

        def st_wait(s):
            pltpu.make_async_copy(avs[s], gs_h.at[pl.ds(0, CH)], ssems[s]).wait()

        def vadd(s):
            a, b = avs[s], bvs[s]

            def rows(i, carry):
                for r in range(4):
                    for j in range(LAT // 16):
                        sl = pl.ds(j * 16, 16)
                        a[4 * i + r, sl] += b[4 * i + r, sl]
                return carry

            lax.fori_loop(0, CH // 4, rows, 0)

        @pl.when(nt >= 1)
        def _():
            idx_start(0, 0)
            idx_wait(0)
            gat_start(0)

        @pl.when(nt >= 2)
        def _():
            idx_start(1, 1)

        def half(t, s):
            @pl.when(t < nt)
            def _():
                o = 1 - s
                gat_wait(s)

                @pl.when(t + 2 < nt)
                def _():
                    idx_start(t + 2, s)

                @pl.when(t + 1 < nt)
                def _():
                    idx_wait(o)

                    @pl.when(t >= 1)
                    def _():
                        st_wait(o)

                    gat_start(o)

                vadd(s)
                st_start(t, s)

        def pair(p, carry):
            half(2 * p, 0)
            half(2 * p + 1, 1)
            return carry

        lax.fori_loop(0, (nt + 1) // 2, pair, 0)

        @pl.when(nt >= 1)
        def _():
            st_wait(0)

        @pl.when(nt >= 2)
        def _():
            st_wait(1)

    return pl.kernel(
        body,
        out_type=jax.ShapeDtypeStruct((e, LAT), F32),
        mesh=mesh,
        scratch_types=[
            pltpu.VMEM((CH,), jnp.int32),
            pltpu.VMEM((CH,), jnp.int32),
            pltpu.VMEM((CH,), jnp.int32),
            pltpu.VMEM((CH,), jnp.int32),
            pltpu.VMEM((CH, LAT), F32),
            pltpu.VMEM((CH, LAT), F32),
            pltpu.VMEM((CH, LAT), F32),
            pltpu.VMEM((CH, LAT), F32),
        ] + [pltpu.SemaphoreType.DMA] * 6,
    )


_NZF = N_NODES // CH
_NZT = N_NODES - _NZF * CH


@functools.lru_cache(maxsize=None)
def _sc_scatter_fn(e, tok_shape):
    nchunk = e // CH
    mesh = plsc.VectorSubcoreMesh(core_axis_name="c", subcore_axis_name="s")

    def body(tok_h, nf_h, dst_h, out_h, acc, zbuf, nb0, nb1, id0, id1,
             psem0, psem1):
        del tok_h
        cc = lax.axis_index("c")
        ss = lax.axis_index("s")

        def zr(i, carry):
            for j in range(LAT // 16):
                zbuf[i, pl.ds(j * 16, 16)] = jnp.zeros((16,), F32)
            return carry

        lax.fori_loop(0, CH, zr, 0)

        def zz(k, carry):
            ch = ss + k * NSUB

            @pl.when(ch < _NZF)
            def _():
                pltpu.sync_copy(zbuf, acc.at[pl.ds(ch * CH, CH)])

            @pl.when(ch == _NZF)
            def _():
                pltpu.sync_copy(zbuf.at[pl.ds(0, _NZT)], acc.at[pl.ds(_NZF * CH, _NZT)])

            return carry

        lax.fori_loop(0, (_NZF + 1 + NSUB - 1) // NSUB, zz, 0)
        plsc.subcore_barrier()

        w = ss * NCORE + cc
        nt = (nchunk - w + NW - 1) // NW

        nbs = (nb0, nb1)
        idv = (id0, id1)
        psems = (psem0, psem1)

        def pf_start(t, s):
            base = (w + t * NW) * CH
            pltpu.async_copy(dst_h.at[pl.ds(base, CH)], idv[s], psems[s])
            pltpu.async_copy(nf_h.at[pl.ds(base, CH)], nbs[s], psems[s])

        def pf_wait(s):
            pltpu.make_async_copy(dst_h.at[pl.ds(0, CH)], idv[s], psems[s]).wait()
            pltpu.make_async_copy(nf_h.at[pl.ds(0, CH)], nbs[s], psems[s]).wait()

        @pl.when(nt >= 1)
        def _():
            pf_start(0, 0)

        @pl.when(nt >= 2)
        def _():
            pf_start(1, 1)

        def half(t, s):
            @pl.when(t < nt)
            def _():
                pf_wait(s)
                pltpu.sync_copy(nbs[s], acc.at[idv[s]], add=True)

                @pl.when(t + 2 < nt)
                def _():
                    pf_start(t + 2, s)

        def pair(p, carry):
            half(2 * p, 0)
            half(2 * p + 1, 1)
            return carry

        lax.fori_loop(0, (nt + 1) // 2, pair, 0)
        plsc.subcore_barrier()

        def wo(k, carry):
            ch = ss + k * NSUB
            obase = cc * N_NODES

            @pl.when(ch < _NZF)
            def _():
                pltpu.sync_copy(acc.at[pl.ds(ch * CH, CH)],
                                out_h.at[pl.ds(obase + ch * CH, CH)])

            @pl.when(ch == _NZF)
            def _():
                pltpu.sync_copy(acc.at[pl.ds(_NZF * CH, _NZT)],
                                out_h.at[pl.ds(obase + _NZF * CH, _NZT)])

            return carry

        lax.fori_loop(0, (_NZF + 1 + NSUB - 1) // NSUB, wo, 0)

    return pl.kernel(
        body,
        out_type=jax.ShapeDtypeStruct((NCORE * N_NODES, LAT), F32),
        mesh=mesh,
        scratch_types=[
            pltpu.VMEM_SHARED((N_NODES, LAT), F32),
            pltpu.VMEM((CH, LAT), F32),
            pltpu.VMEM((CH, LAT), F32),
            pltpu.VMEM((CH, LAT), F32),
            pltpu.VMEM((CH,), jnp.int32),
            pltpu.VMEM((CH,), jnp.int32),
            pltpu.SemaphoreType.DMA,
            pltpu.SemaphoreType.DMA,
        ],
    )


def kernel(node_features, mesh_edge_features, world_edge_features, coarse_edge_features,
           mesh_edge_index, world_edge_index, coarse_edge_index, params):
    feats_in = {'mesh': mesh_edge_features, 'world': world_edge_features,
                'coarse': coarse_edge_features}
    idx = {'mesh': mesh_edge_index, 'world': world_edge_index,
           'coarse': coarse_edge_index}
    src = {ek: idx[ek][0].astype(jnp.int32) for ek in _EKEYS}
    dst = {ek: idx[ek][1].astype(jnp.int32) for ek in _EKEYS}

    x = _mlp_ln(node_features, *_unpack_mlp(params['node_encoder']))
    sc_tok = x
    feats = {}

    _ORDER = ('mesh', 'world', 'coarse')

    nsteps = len(params['blocks'])
    for si, blk in enumerate(params['blocks']):
        pws = []
        esplit = {}
        for ek in _EKEYS:
            w0 = blk['edge'][ek]['layers'][0]['W']
            esplit[ek] = w0[:LAT]
            pws.append(w0[LAT:2 * LAT])
            pws.append(w0[2 * LAT:])
        proj = _proj6(x, pws)
        tabs = {ek: (proj[2 * i], proj[2 * i + 1]) for i, ek in enumerate(_EKEYS)}

        gsums = {}
        for ek in _ORDER:
            e = src[ek].shape[0]
            gsums[ek] = _sc_gather_fn(e, sc_tok.shape)(
                sc_tok, tabs[ek][0], tabs[ek][1], src[ek], dst[ek])
            sc_tok = gsums[ek]

        nfs = {}
        new_feats = {}
        want_fnew = si + 1 < nsteps
        for ek in _ORDER:
            p = blk['edge'][ek]
            b0 = p['layers'][0]['b'].reshape(1, -1)
            w1 = p['layers'][1]['W']
            b1 = p['layers'][1]['b'].reshape(1, -1)
            w2 = p['layers'][2]['W']
            b2 = p['layers'][2]['b'].reshape(1, -1)
            g = p['ln']['g'].reshape(1, -1)
            bb = p['ln']['b'].reshape(1, -1)
            if si == 0:
                nf, fnew = _edge_update_enc(
                    gsums[ek], feats_in[ek].T,
                    tuple(_unpack_mlp(params['edge_encoders'][ek])),
                    esplit[ek], b0, w1, b1, w2, b2, g, bb)
                nfs[ek] = nf
                new_feats[ek] = fnew
            else:
                outs = _edge_update(gsums[ek], feats[ek], esplit[ek], b0,
                                    w1, b1, w2, b2, g, bb, want_fnew)
                nfs[ek] = outs[0]
                if want_fnew:
                    new_feats[ek] = outs[1]

        agg_by_key = {}
        for ek in _ORDER:
            agg = _sc_scatter_fn(nfs[ek].shape[0], sc_tok.shape)(
                sc_tok, nfs[ek], dst[ek])
            sc_tok = agg
            agg_by_key[ek] = agg
        aggs = [agg_by_key[ek] for ek in _EKEYS]

        np_ = blk['node']
        w0n = np_['layers'][0]['W']
        wx, wc, wm, wv = w0n[:LAT], w0n[LAT:2 * LAT], w0n[2 * LAT:3 * LAT], w0n[3 * LAT:]
        x = _node_update(
            x, aggs, wx, wc, wm, wv,
            np_['layers'][0]['b'].reshape(1, -1),
            np_['layers'][1]['W'], np_['layers'][1]['b'].reshape(1, -1),
            np_['layers'][2]['W'], np_['layers'][2]['b'].reshape(1, -1),
            np_['ln']['g'].reshape(1, -1), np_['ln']['b'].reshape(1, -1))
        if si + 1 < nsteps:
            feats = new_feats

    dp = params['decoder']
    return _decode(x, dp['layers'][0]['W'], dp['layers'][0]['b'].reshape(1, -1),
                   dp['layers'][1]['W'], dp['layers'][1]['b'].reshape(1, -1),
                   dp['layers'][2]['W'], dp['layers'][2]['b'].reshape(1, -1))


# speedup vs baseline: 1.1869x; 1.1869x over previous
import functools

import jax
import jax.numpy as jnp
from jax import lax
from jax.experimental import pallas as pl
from jax.experimental.pallas import tpu as pltpu
from jax.experimental.pallas import tpu_sc as plsc

F32 = jnp.float32
LAT = 128
N_NODES = 10000
NCORE = 2
NSUB = 16
NW = NCORE * NSUB
CH = 128
BR = 2000

_EKEYS = ('coarse', 'mesh', 'world')


def _ln_apply(y, g, b):
    m = jnp.mean(y, axis=-1, keepdims=True)
    v = jnp.mean((y - m) ** 2, axis=-1, keepdims=True)
    return (y - m) * lax.rsqrt(v + 1e-5) * g + b


def _unpack_mlp(p):
    ls = p['layers']
    out = []
    for l in ls:
        out.append(l['W'])
        out.append(l['b'].reshape(1, -1))
    if 'ln' in p:
        out.append(p['ln']['g'].reshape(1, -1))
        out.append(p['ln']['b'].reshape(1, -1))
    return out


def _wspec(shape):
    return pl.BlockSpec(shape, lambda i: (0, 0))


def _mlp_ln(x, w0, b0, w1, b1, w2, b2, g, bb):
    n, din = x.shape

    def body(x_r, w0_r, b0_r, w1_r, b1_r, w2_r, b2_r, g_r, bb_r, o_r):
        h = jnp.maximum(jnp.dot(x_r[...], w0_r[...], preferred_element_type=F32) + b0_r[...], 0.0)
        h = jnp.maximum(jnp.dot(h, w1_r[...], preferred_element_type=F32) + b1_r[...], 0.0)
        y = jnp.dot(h, w2_r[...], preferred_element_type=F32) + b2_r[...]
        o_r[...] = _ln_apply(y, g_r[...], bb_r[...])

    return pl.pallas_call(
        body,
        grid=(n // BR,),
        in_specs=[
            pl.BlockSpec((BR, din), lambda i: (i, 0)),
            _wspec((din, LAT)), _wspec((1, LAT)),
            _wspec((LAT, LAT)), _wspec((1, LAT)),
            _wspec((LAT, LAT)), _wspec((1, LAT)),
            _wspec((1, LAT)), _wspec((1, LAT)),
        ],
        out_specs=pl.BlockSpec((BR, LAT), lambda i: (i, 0)),
        out_shape=jax.ShapeDtypeStruct((n, LAT), F32),
    )(x, w0, b0, w1, b1, w2, b2, g, bb)


def _proj6(x, ws):
    n = x.shape[0]

    def body(x_r, *refs):
        xv = x_r[...]
        for k in range(6):
            refs[6 + k][...] = jnp.dot(xv, refs[k][...], preferred_element_type=F32)

    return pl.pallas_call(
        body,
        grid=(n // BR,),
        in_specs=[pl.BlockSpec((BR, LAT), lambda i: (i, 0))] + [_wspec((LAT, LAT))] * 6,
        out_specs=[pl.BlockSpec((BR, LAT), lambda i: (i, 0))] * 6,
        out_shape=[jax.ShapeDtypeStruct((n, LAT), F32)] * 6,
    )(x, *ws)


def _edge_update_enc(gsum, raw, encp, w0f, b0, w1, b1, w2, b2, g, bb):
    e, din = raw.shape
    ew0, eb0, ew1, eb1, ew2, eb2, eg, ebb = encp

    def body(gs_r, raw_r, ew0_r, eb0_r, ew1_r, eb1_r, ew2_r, eb2_r, eg_r, ebb_r,
             w0_r, b0_r, w1_r, b1_r, w2_r, b2_r, g_r, bb_r, nf_o, fn_o):
        h = jnp.maximum(jnp.dot(raw_r[...], ew0_r[...], preferred_element_type=F32) + eb0_r[...], 0.0)
        h = jnp.maximum(jnp.dot(h, ew1_r[...], preferred_element_type=F32) + eb1_r[...], 0.0)
        y = jnp.dot(h, ew2_r[...], preferred_element_type=F32) + eb2_r[...]
        fv = _ln_apply(y, eg_r[...], ebb_r[...])
        h = gs_r[...] + jnp.dot(fv, w0_r[...], preferred_element_type=F32) + b0_r[...]
        h = jnp.maximum(h, 0.0)
        h = jnp.maximum(jnp.dot(h, w1_r[...], preferred_element_type=F32) + b1_r[...], 0.0)
        y = jnp.dot(h, w2_r[...], preferred_element_type=F32) + b2_r[...]
        nf = _ln_apply(y, g_r[...], bb_r[...])
        nf_o[...] = nf
        fn_o[...] = fv + nf

    return pl.pallas_call(
        body,
        grid=(e // BR,),
        in_specs=[pl.BlockSpec((BR, LAT), lambda i: (i, 0)),
                  pl.BlockSpec((BR, din), lambda i: (i, 0)),
                  _wspec((din, LAT)), _wspec((1, LAT)),
                  _wspec((LAT, LAT)), _wspec((1, LAT)),
                  _wspec((LAT, LAT)), _wspec((1, LAT)),
                  _wspec((1, LAT)), _wspec((1, LAT)),
                  _wspec((LAT, LAT)), _wspec((1, LAT)),
                  _wspec((LAT, LAT)), _wspec((1, LAT)),
                  _wspec((LAT, LAT)), _wspec((1, LAT)),
                  _wspec((1, LAT)), _wspec((1, LAT))],
        out_specs=[pl.BlockSpec((BR, LAT), lambda i: (i, 0))] * 2,
        out_shape=[jax.ShapeDtypeStruct((e, LAT), F32)] * 2,
    )(gsum, raw, *encp, w0f, b0, w1, b1, w2, b2, g, bb)


def _edge_update(gsum, feats, w0f, b0, w1, b1, w2, b2, g, bb, want_fnew):
    e = feats.shape[0]

    def body(gs_r, f_r, w0_r, b0_r, w1_r, b1_r, w2_r, b2_r, g_r, bb_r, *outs):
        fv = f_r[...]
        h = gs_r[...] + jnp.dot(fv, w0_r[...], preferred_element_type=F32) + b0_r[...]
        h = jnp.maximum(h, 0.0)
        h = jnp.maximum(jnp.dot(h, w1_r[...], preferred_element_type=F32) + b1_r[...], 0.0)
        y = jnp.dot(h, w2_r[...], preferred_element_type=F32) + b2_r[...]
        nf = _ln_apply(y, g_r[...], bb_r[...])
        outs[0][...] = nf
        if want_fnew:
            outs[1][...] = fv + nf

    n_out = 2 if want_fnew else 1
    return pl.pallas_call(
        body,
        grid=(e // BR,),
        in_specs=[pl.BlockSpec((BR, LAT), lambda i: (i, 0))] * 2 + [
            _wspec((LAT, LAT)), _wspec((1, LAT)),
            _wspec((LAT, LAT)), _wspec((1, LAT)),
            _wspec((LAT, LAT)), _wspec((1, LAT)),
            _wspec((1, LAT)), _wspec((1, LAT)),
        ],
        out_specs=[pl.BlockSpec((BR, LAT), lambda i: (i, 0))] * n_out,
        out_shape=[jax.ShapeDtypeStruct((e, LAT), F32)] * n_out,
    )(gsum, feats, w0f, b0, w1, b1, w2, b2, g, bb)


def _node_update(x, aggs, wx, wc, wm, ww, b0, w1, b1, w2, b2, g, bb):
    n = x.shape[0]
    nb = N_NODES // BR

    def body(x_r, c0_r, c1_r, m0_r, m1_r, v0_r, v1_r,
             wx_r, wc_r, wm_r, ww_r, b0_r, w1_r, b1_r, w2_r, b2_r, g_r, bb_r, o_r):
        xv = x_r[...]
        h = jnp.dot(xv, wx_r[...], preferred_element_type=F32)
        h += jnp.dot(c0_r[...] + c1_r[...], wc_r[...], preferred_element_type=F32)
        h += jnp.dot(m0_r[...] + m1_r[...], wm_r[...], preferred_element_type=F32)
        h += jnp.dot(v0_r[...] + v1_r[...], ww_r[...], preferred_element_type=F32)
        h = jnp.maximum(h + b0_r[...], 0.0)
        h = jnp.maximum(jnp.dot(h, w1_r[...], preferred_element_type=F32) + b1_r[...], 0.0)
        y = jnp.dot(h, w2_r[...], preferred_element_type=F32) + b2_r[...]
        o_r[...] = xv + _ln_apply(y, g_r[...], bb_r[...])

    part_specs = []
    for _ in range(3):
        part_specs.append(pl.BlockSpec((BR, LAT), lambda i: (i, 0)))
        part_specs.append(pl.BlockSpec((BR, LAT), lambda i, _nb=nb: (i + _nb, 0)))
    return pl.pallas_call(
        body,
        grid=(n // BR,),
        in_specs=[pl.BlockSpec((BR, LAT), lambda i: (i, 0))] + part_specs + [
            _wspec((LAT, LAT)), _wspec((LAT, LAT)), _wspec((LAT, LAT)), _wspec((LAT, LAT)),
            _wspec((1, LAT)),
            _wspec((LAT, LAT)), _wspec((1, LAT)),
            _wspec((LAT, LAT)), _wspec((1, LAT)),
            _wspec((1, LAT)), _wspec((1, LAT)),
        ],
        out_specs=pl.BlockSpec((BR, LAT), lambda i: (i, 0)),
        out_shape=jax.ShapeDtypeStruct((n, LAT), F32),
    )(x, aggs[0], aggs[0], aggs[1], aggs[1], aggs[2], aggs[2],
      wx, wc, wm, ww, b0, w1, b1, w2, b2, g, bb)


def _decode(x, w0, b0, w1, b1, w2, b2):
    n = x.shape[0]
    dout = w2.shape[1]

    def body(x_r, w0_r, b0_r, w1_r, b1_r, w2_r, b2_r, o_r):
        h = jnp.maximum(jnp.dot(x_r[...], w0_r[...], preferred_element_type=F32) + b0_r[...], 0.0)
        h = jnp.maximum(jnp.dot(h, w1_r[...], preferred_element_type=F32) + b1_r[...], 0.0)
        o_r[...] = jnp.dot(h, w2_r[...], preferred_element_type=F32) + b2_r[...]

    return pl.pallas_call(
        body,
        grid=(n // BR,),
        in_specs=[
            pl.BlockSpec((BR, LAT), lambda i: (i, 0)),
            _wspec((LAT, LAT)), _wspec((1, LAT)),
            _wspec((LAT, LAT)), _wspec((1, LAT)),
            _wspec((LAT, dout)), _wspec((1, dout)),
        ],
        out_specs=pl.BlockSpec((BR, dout), lambda i: (i, 0)),
        out_shape=jax.ShapeDtypeStruct((n, dout), F32),
    )(x, w0, b0, w1, b1, w2, b2)


@functools.lru_cache(maxsize=None)
def _sc_gather_fn(e, tok_shape):
    nchunk = e // CH
    mesh = plsc.VectorSubcoreMesh(core_axis_name="c", subcore_axis_name="s")

    def body(tok_h, xs_h, xd_h, src_h, dst_h, gs_h,
             ix0, ix1, id0, id1, a0, a1, b0, b1,
             isem0, isem1, gsem0, gsem1, ssem0, ssem1):
        del tok_h
        cc = lax.axis_index("c")
        ss = lax.axis_index("s")
        w = ss * NCORE + cc
        nt = (nchunk - w + NW - 1) // NW

        ixs = (ix0, ix1)
        ids = (id0, id1)
        avs = (a0, a1)
        bvs = (b0, b1)
        isems = (isem0, isem1)
        gsems = (gsem0, gsem1)
        ssems = (ssem0, ssem1)

        def idx_start(t, s):
            base = (w + t * NW) * CH
            pltpu.async_copy(src_h.at[pl.ds(base, CH)], ixs[s], isems[s])
            pltpu.async_copy(dst_h.at[pl.ds(base, CH)], ids[s], isems[s])

        def idx_wait(s):
            pltpu.make_async_copy(src_h.at[pl.ds(0, CH)], ixs[s], isems[s]).wait()
            pltpu.make_async_copy(dst_h.at[pl.ds(0, CH)], ids[s], isems[s]).wait()

        def gat_start(s):
            pltpu.async_copy(xs_h.at[ixs[s]], avs[s], gsems[s])
            pltpu.async_copy(xd_h.at[ids[s]], bvs[s], gsems[s])

        def gat_wait(s):
            pltpu.make_async_copy(xs_h.at[ixs[s]], avs[s], gsems[s]).wait()
            pltpu.make_async_copy(xd_h.at[ids[s]], bvs[s], gsems[s]).wait()

        def st_start(t, s):
            base = (w + t * NW) * CH
            pltpu.async_copy(avs[s], gs_h.at[pl.ds(base, CH)], ssems[s])

        def st_wait(s):
            pltpu.make_async_copy(avs[s], gs_h.at[pl.ds(0, CH)], ssems[s]).wait()

        def vadd(s):
            a, b = avs[s], bvs[s]

            def rows(i, carry):
                for r in range(4):
                    for j in range(LAT // 16):
                        sl = pl.ds(j * 16, 16)
                        a[4 * i + r, sl] += b[4 * i + r, sl]
                return carry

            lax.fori_loop(0, CH // 4, rows, 0)

        @pl.when(nt >= 1)
        def _():
            idx_start(0, 0)
            idx_wait(0)
            gat_start(0)

        @pl.when(nt >= 2)
        def _():
            idx_start(1, 1)

        def half(t, s):
            @pl.when(t < nt)
            def _():
                o = 1 - s
                gat_wait(s)

                @pl.when(t + 2 < nt)
                def _():
                    idx_start(t + 2, s)

                @pl.when(t + 1 < nt)
                def _():
                    idx_wait(o)

                    @pl.when(t >= 1)
                    def _():
                        st_wait(o)

                    gat_start(o)

                vadd(s)
                st_start(t, s)

        def pair(p, carry):
            half(2 * p, 0)
            half(2 * p + 1, 1)
            return carry

        lax.fori_loop(0, (nt + 1) // 2, pair, 0)

        @pl.when(nt >= 1)
        def _():
            st_wait(0)

        @pl.when(nt >= 2)
        def _():
            st_wait(1)

    return pl.kernel(
        body,
        out_type=jax.ShapeDtypeStruct((e, LAT), F32),
        mesh=mesh,
        scratch_types=[
            pltpu.VMEM((CH,), jnp.int32),
            pltpu.VMEM((CH,), jnp.int32),
            pltpu.VMEM((CH,), jnp.int32),
            pltpu.VMEM((CH,), jnp.int32),
            pltpu.VMEM((CH, LAT), F32),
            pltpu.VMEM((CH, LAT), F32),
            pltpu.VMEM((CH, LAT), F32),
            pltpu.VMEM((CH, LAT), F32),
        ] + [pltpu.SemaphoreType.DMA] * 6,
    )


_NZF = N_NODES // CH
_NZT = N_NODES - _NZF * CH


@functools.lru_cache(maxsize=None)
def _sc_scatter_fn(e, tok_shape):
    nchunk = e // CH
    mesh = plsc.VectorSubcoreMesh(core_axis_name="c", subcore_axis_name="s")

    def body(tok_h, nf_h, dst_h, out_h, acc, zbuf, nb0, nb1, id0, id1,
             psem0, psem1):
        del tok_h
        cc = lax.axis_index("c")
        ss = lax.axis_index("s")

        def zr(i, carry):
            for j in range(LAT // 16):
                zbuf[i, pl.ds(j * 16, 16)] = jnp.zeros((16,), F32)
            return carry

        lax.fori_loop(0, CH, zr, 0)

        def zz(k, carry):
            ch = ss + k * NSUB

            @pl.when(ch < _NZF)
            def _():
                pltpu.sync_copy(zbuf, acc.at[pl.ds(ch * CH, CH)])

            @pl.when(ch == _NZF)
            def _():
                pltpu.sync_copy(zbuf.at[pl.ds(0, _NZT)], acc.at[pl.ds(_NZF * CH, _NZT)])

            return carry

        lax.fori_loop(0, (_NZF + 1 + NSUB - 1) // NSUB, zz, 0)
        plsc.subcore_barrier()

        w = ss * NCORE + cc
        nt = (nchunk - w + NW - 1) // NW

        nbs = (nb0, nb1)
        idv = (id0, id1)
        psems = (psem0, psem1)

        def pf_start(t, s):
            base = (w + t * NW) * CH
            pltpu.async_copy(dst_h.at[pl.ds(base, CH)], idv[s], psems[s])
            pltpu.async_copy(nf_h.at[pl.ds(base, CH)], nbs[s], psems[s])

        def pf_wait(s):
            pltpu.make_async_copy(dst_h.at[pl.ds(0, CH)], idv[s], psems[s]).wait()
            pltpu.make_async_copy(nf_h.at[pl.ds(0, CH)], nbs[s], psems[s]).wait()

        @pl.when(nt >= 1)
        def _():
            pf_start(0, 0)

        @pl.when(nt >= 2)
        def _():
            pf_start(1, 1)

        def half(t, s):
            @pl.when(t < nt)
            def _():
                pf_wait(s)
                pltpu.sync_copy(nbs[s], acc.at[idv[s]], add=True)

                @pl.when(t + 2 < nt)
                def _():
                    pf_start(t + 2, s)

        def pair(p, carry):
            half(2 * p, 0)
            half(2 * p + 1, 1)
            return carry

        lax.fori_loop(0, (nt + 1) // 2, pair, 0)
        plsc.subcore_barrier()

        def wo(k, carry):
            ch = ss + k * NSUB
            obase = cc * N_NODES

            @pl.when(ch < _NZF)
            def _():
                pltpu.sync_copy(acc.at[pl.ds(ch * CH, CH)],
                                out_h.at[pl.ds(obase + ch * CH, CH)])

            @pl.when(ch == _NZF)
            def _():
                pltpu.sync_copy(acc.at[pl.ds(_NZF * CH, _NZT)],
                                out_h.at[pl.ds(obase + _NZF * CH, _NZT)])

            return carry

        lax.fori_loop(0, (_NZF + 1 + NSUB - 1) // NSUB, wo, 0)

    return pl.kernel(
        body,
        out_type=jax.ShapeDtypeStruct((NCORE * N_NODES, LAT), F32),
        mesh=mesh,
        scratch_types=[
            pltpu.VMEM_SHARED((N_NODES, LAT), F32),
            pltpu.VMEM((CH, LAT), F32),
            pltpu.VMEM((CH, LAT), F32),
            pltpu.VMEM((CH, LAT), F32),
            pltpu.VMEM((CH,), jnp.int32),
            pltpu.VMEM((CH,), jnp.int32),
            pltpu.SemaphoreType.DMA,
            pltpu.SemaphoreType.DMA,
        ],
    )


def kernel(node_features, mesh_edge_features, world_edge_features, coarse_edge_features,
           mesh_edge_index, world_edge_index, coarse_edge_index, params):
    feats_in = {'mesh': mesh_edge_features, 'world': world_edge_features,
                'coarse': coarse_edge_features}
    idx = {'mesh': mesh_edge_index, 'world': world_edge_index,
           'coarse': coarse_edge_index}
    src = {ek: idx[ek][0].astype(jnp.int32) for ek in _EKEYS}
    dst = {ek: idx[ek][1].astype(jnp.int32) for ek in _EKEYS}

    x = _mlp_ln(node_features, *_unpack_mlp(params['node_encoder']))
    sc_tok = x
    feats = {}

    _ORDER = ('mesh', 'world', 'coarse')

    nsteps = len(params['blocks'])
    for si, blk in enumerate(params['blocks']):
        pws = []
        esplit = {}
        for ek in _EKEYS:
            w0 = blk['edge'][ek]['layers'][0]['W']
            esplit[ek] = w0[:LAT]
            pws.append(w0[LAT:2 * LAT])
            pws.append(w0[2 * LAT:])
        proj = _proj6(x, pws)
        tabs = {ek: (proj[2 * i], proj[2 * i + 1]) for i, ek in enumerate(_EKEYS)}

        gsums = {}
        for ek in _ORDER:
            e = src[ek].shape[0]
            gsums[ek] = _sc_gather_fn(e, sc_tok.shape)(
                sc_tok, tabs[ek][0], tabs[ek][1], src[ek], dst[ek])
            sc_tok = gsums[ek]

        nfs = {}
        new_feats = {}
        want_fnew = si + 1 < nsteps
        for ek in _ORDER:
            p = blk['edge'][ek]
            b0 = p['layers'][0]['b'].reshape(1, -1)
            w1 = p['layers'][1]['W']
            b1 = p['layers'][1]['b'].reshape(1, -1)
            w2 = p['layers'][2]['W']
            b2 = p['layers'][2]['b'].reshape(1, -1)
            g = p['ln']['g'].reshape(1, -1)
            bb = p['ln']['b'].reshape(1, -1)
            if si == 0:
                nf, fnew = _edge_update_enc(
                    gsums[ek], feats_in[ek],
                    tuple(_unpack_mlp(params['edge_encoders'][ek])),
                    esplit[ek], b0, w1, b1, w2, b2, g, bb)
                nfs[ek] = nf
                new_feats[ek] = fnew
            else:
                outs = _edge_update(gsums[ek], feats[ek], esplit[ek], b0,
                                    w1, b1, w2, b2, g, bb, want_fnew)
                nfs[ek] = outs[0]
                if want_fnew:
                    new_feats[ek] = outs[1]

        agg_by_key = {}
        for ek in _ORDER:
            agg = _sc_scatter_fn(nfs[ek].shape[0], sc_tok.shape)(
                sc_tok, nfs[ek], dst[ek])
            sc_tok = agg
            agg_by_key[ek] = agg
        aggs = [agg_by_key[ek] for ek in _EKEYS]

        np_ = blk['node']
        w0n = np_['layers'][0]['W']
        wx, wc, wm, wv = w0n[:LAT], w0n[LAT:2 * LAT], w0n[2 * LAT:3 * LAT], w0n[3 * LAT:]
        x = _node_update(
            x, aggs, wx, wc, wm, wv,
            np_['layers'][0]['b'].reshape(1, -1),
            np_['layers'][1]['W'], np_['layers'][1]['b'].reshape(1, -1),
            np_['layers'][2]['W'], np_['layers'][2]['b'].reshape(1, -1),
            np_['ln']['g'].reshape(1, -1), np_['ln']['b'].reshape(1, -1))
        if si + 1 < nsteps:
            feats = new_feats

    dp = params['decoder']
    return _decode(x, dp['layers'][0]['W'], dp['layers'][0]['b'].reshape(1, -1),
                   dp['layers'][1]['W'], dp['layers'][1]['b'].reshape(1, -1),
                   dp['layers'][2]['W'], dp['layers'][2]['b'].reshape(1, -1))
